# baseline (speedup 1.0000x reference)
import jax
import jax.numpy as jnp
from jax import lax
from jax.experimental import pallas as pl
from jax.experimental.pallas import tpu as pltpu
from jax.experimental.pallas import tpu_sc as plsc

B = 16384
DIM = 64
NC = 2
NS = 16
NW = NC * NS
BPW = B // NW
CH = 64
NCH = BPW // CH
LANES = 16

WCOLS = 256
EBLK = 8594
UBLK = 782
EB_W = -(-EBLK // NW)
UB_W = -(-UBLK // NW)
TWIN_E = -(-(EB_W * 128) // WCOLS)
TWIN_U = -(-(UB_W * 128) // WCOLS)
MYMAX = 2048
WMAX = 96
NE = 2 * B
NU = B
SROWS_E = NE + LANES
SROWS_U = NU + LANES

_LOG1P_COEF = (
    5.621959008883515e-07, 0.999957487075066, -0.49920656854784484,
    0.3269731000138668, -0.22283625832801954, 0.1307650325042385,
    -0.052624851367851076, 0.010119082927824848,
)


def _log1p_poly(t):
    acc = jnp.full_like(t, _LOG1P_COEF[-1])
    for c in reversed(_LOG1P_COEF[:-1]):
        acc = acc * t + jnp.float32(c)
    return acc


def _phase1_body(pos_hbm, neg_hbm, users_hbm, etab_t, utab_t,
                 estage, ustage,
                 piv, niv, uiv, myr, myp, winr, winp,
                 winbuf, rowbuf, wsem, ssem):
    wid = lax.axis_index("s") * NC + lax.axis_index("c")
    lane = lax.iota(jnp.int32, LANES)

    pltpu.sync_copy(pos_hbm, piv)
    pltpu.sync_copy(neg_hbm, niv)
    pltpu.sync_copy(users_hbm, uiv)

    def run(tab_t, nblk, blk_w, twin, idx_refs, stage_out, dummy_row):
        lo_w = wid * blk_w * 128
        hi_w = jnp.minimum((wid + 1) * blk_w, nblk) * 128

        ofs0 = 0
        for ref, slot0 in idx_refs:
            def l1(cchunk, ofs, _ref=ref, _slot0=slot0):
                v = _ref[pl.ds(cchunk * LANES, LANES)]
                m = (v >= lo_w) & (v < hi_w)
                sl = pl.ds(ofs, LANES)
                plsc.store_compressed(myr.at[sl], v, mask=m)
                plsc.store_compressed(
                    myp.at[sl], cchunk * LANES + lane + _slot0, mask=m)
                return ofs + jnp.max(plsc.all_reduce_population_count(m))
            ofs0 = lax.fori_loop(0, B // LANES, l1, ofs0)
        nmine = ofs0

        def issue(t):
            @pl.when(lo_w + t * WCOLS < hi_w)
            def _():
                lo = pl.multiple_of(wid * blk_w * 128 + t * WCOLS, 128)
                pltpu.async_copy(tab_t.at[:, pl.ds(lo, WCOLS)],
                                 winbuf.at[t % 2], wsem)

        def wait_win(slot):
            pltpu.make_async_copy(tab_t.at[:, pl.ds(0, WCOLS)],
                                  winbuf.at[slot], wsem).wait()

        def drain_one():
            pltpu.make_async_copy(stage_out.at[pl.ds(0, LANES), :],
                                  rowbuf.at[0], ssem).wait()

        issue(0)

        def win_body(t, gtot):
            lo = lo_w + t * WCOLS
            valid = lo < hi_w

            @pl.when(valid)
            def _():
                wait_win(t % 2)
            issue(t + 1)

            def l2(cchunk, ofs):
                sl_in = pl.ds(cchunk * LANES, LANES)
                v = myr[sl_in]
                p = myp[sl_in]
                inb = (cchunk * LANES + lane) < nmine
                m = (v >= lo) & (v < lo + WCOLS) & inb
                sl = pl.ds(ofs, LANES)
                plsc.store_compressed(winr.at[sl], v, mask=m)
                plsc.store_compressed(winp.at[sl], p, mask=m)
                return ofs + jnp.max(plsc.all_reduce_population_count(m))

            cnt = lax.cond(
                valid,
                lambda: lax.fori_loop(0, MYMAX // LANES, l2, 0),
                lambda: 0)

            winr[pl.ds(cnt, LANES)] = jnp.full((LANES,), lo, jnp.int32)
            winp[pl.ds(cnt, LANES)] = jnp.full((LANES,), dummy_row,
                                               jnp.int32)

            def grp(kk, g):
                sl = pl.ds(kk * LANES, LANES)
                roff = winr[sl] - lo
                pvec = winp[sl]
                rslot = g & 1

                @pl.when(g >= 2)
                def _():
                    drain_one()
                for d in range(DIM):
                    vals = plsc.load_gather(
                        winbuf.at[t % 2],
                        [jnp.full((LANES,), d, jnp.int32), roff])
                    plsc.store_scatter(
                        rowbuf.at[rslot],
                        [lane, jnp.full((LANES,), d, jnp.int32)], vals)
                pltpu.async_copy(rowbuf.at[rslot], stage_out.at[pvec], ssem)
                return g + 1

            ngrp = (cnt + LANES - 1) // LANES
            return lax.fori_loop(0, ngrp, grp, gtot)

        gtot = lax.fori_loop(0, twin, win_body, 0)

        def final_drain(k, _):
            drain_one()
            return 0
        lax.fori_loop(0, jnp.minimum(gtot, 2), final_drain, 0)

    run(etab_t, EBLK, EB_W, TWIN_E,
        [(piv, 0), (niv, B)], estage, NE)
    run(utab_t, UBLK, UB_W, TWIN_U,
        [(uiv, 0)], ustage, NU)


def _phase2_body(estage, ustage, out_hbm,
                 ubuf, pbuf, nbuf, part, usem, psem, nsem):
    wid = lax.axis_index("s") * NC + lax.axis_index("c")
    base = wid * BPW

    def fire(c, slot):
        off = base + c * CH
        pltpu.async_copy(ustage.at[pl.ds(off, CH), :], ubuf.at[slot], usem)
        pltpu.async_copy(estage.at[pl.ds(off, CH), :], pbuf.at[slot], psem)
        pltpu.async_copy(estage.at[pl.ds(B + off, CH), :], nbuf.at[slot],
                         nsem)

    def drain(slot):
        pltpu.make_async_copy(ustage.at[pl.ds(0, CH), :],
                              ubuf.at[slot], usem).wait()
        pltpu.make_async_copy(estage.at[pl.ds(0, CH), :],
                              pbuf.at[slot], psem).wait()
        pltpu.make_async_copy(estage.at[pl.ds(0, CH), :],
                              nbuf.at[slot], nsem).wait()

    def chunk_sum(slot, acc):
        def row_body(i, a):
            for j in range(DIM // LANES):
                sl = pl.ds(j * LANES, LANES)
                u = ubuf[slot, i, sl]
                p = pbuf[slot, i, sl]
                n = nbuf[slot, i, sl]
                ps = u * p
                ns = u * n
                ea = jnp.exp(-jnp.abs(ps))
                eb = jnp.exp(-jnp.abs(ns))
                a = a + (jnp.maximum(-ps, 0.0) + jnp.maximum(ns, 0.0)
                         + _log1p_poly(ea) + _log1p_poly(eb))
            return a
        return lax.fori_loop(0, CH, row_body, acc)

    acc = jnp.zeros((LANES,), jnp.float32)
    fire(0, 0)
    for c in range(NCH):
        drain(c % 2)
        if c + 1 < NCH:
            fire(c + 1, (c + 1) % 2)
        acc = chunk_sum(c % 2, acc)

    part[...] = acc
    pltpu.sync_copy(part, out_hbm.at[wid])


@jax.jit
def _sc_loss(users, pos, neg, utab, etab):
    mesh = plsc.VectorSubcoreMesh(core_axis_name="c", subcore_axis_name="s")

    phase1 = pl.kernel(
        _phase1_body,
        out_type=(jax.ShapeDtypeStruct((SROWS_E, 128), jnp.float32),
                  jax.ShapeDtypeStruct((SROWS_U, 128), jnp.float32)),
        mesh=mesh,
        compiler_params=pltpu.CompilerParams(needs_layout_passes=False),
        scratch_types=[
            pltpu.VMEM((B,), jnp.int32),
            pltpu.VMEM((B,), jnp.int32),
            pltpu.VMEM((B,), jnp.int32),
            pltpu.VMEM((MYMAX + LANES,), jnp.int32),
            pltpu.VMEM((MYMAX + LANES,), jnp.int32),
            pltpu.VMEM((WMAX + LANES,), jnp.int32),
            pltpu.VMEM((WMAX + LANES,), jnp.int32),
            pltpu.VMEM((2, DIM, WCOLS), jnp.float32),
            pltpu.VMEM((2, LANES, 128), jnp.float32),
            pltpu.SemaphoreType.DMA,
            pltpu.SemaphoreType.DMA,
        ],
    )
    estage, ustage = phase1(pos, neg, users, etab.T, utab.T)

    phase2 = pl.kernel(
        _phase2_body,
        out_type=jax.ShapeDtypeStruct((NW, LANES), jnp.float32),
        mesh=mesh,
        compiler_params=pltpu.CompilerParams(needs_layout_passes=False),
        scratch_types=[
            pltpu.VMEM((2, CH, 128), jnp.float32),
            pltpu.VMEM((2, CH, 128), jnp.float32),
            pltpu.VMEM((2, CH, 128), jnp.float32),
            pltpu.VMEM((LANES,), jnp.float32),
            pltpu.SemaphoreType.DMA,
            pltpu.SemaphoreType.DMA,
            pltpu.SemaphoreType.DMA,
        ],
    )
    parts = phase2(estage, ustage)
    return jnp.sum(parts) / jnp.float32(B * DIM)


def kernel(users, pos, neg, user_table, entity_table):
    return _sc_loss(users.astype(jnp.int32), pos.astype(jnp.int32),
                    neg.astype(jnp.int32), user_table, entity_table)


# speedup vs baseline: 2.7168x; 2.7168x over previous
import jax
import jax.numpy as jnp
from jax import lax
from jax.experimental import pallas as pl
from jax.experimental.pallas import tpu as pltpu
from jax.experimental.pallas import tpu_sc as plsc

B = 16384
DIM = 64
ROWP = 128
NC = 2
NS = 16
NW = NC * NS
BPW = B // NW
CH = 64
NCH = BPW // CH
LANES = 16

_LOG1P_COEF = (
    5.621959008883515e-07, 0.999957487075066, -0.49920656854784484,
    0.3269731000138668, -0.22283625832801954, 0.1307650325042385,
    -0.052624851367851076, 0.010119082927824848,
)


def _log1p_poly(t):
    acc = jnp.full_like(t, _LOG1P_COEF[-1])
    for c in reversed(_LOG1P_COEF[:-1]):
        acc = acc * t + jnp.float32(c)
    return acc


def _sc_body(users_hbm, pos_hbm, neg_hbm, utab_hbm, etab_hbm, out_hbm,
             uiv, piv, niv, ubuf, pbuf, nbuf, upar, ppar, npar, part,
             usem, psem, nsem):
    wid = lax.axis_index("s") * NC + lax.axis_index("c")
    base = wid * BPW

    pltpu.sync_copy(users_hbm.at[pl.ds(base, BPW)], uiv)
    pltpu.sync_copy(pos_hbm.at[pl.ds(base, BPW)], piv)
    pltpu.sync_copy(neg_hbm.at[pl.ds(base, BPW)], niv)

    lane = lax.iota(jnp.int32, LANES)

    def fire(c, slot):
        def enq(i, _):
            g = c * CH + (i & ~(LANES - 1))
            k = i & (LANES - 1)
            sel = lane == k
            ru = jnp.sum(jnp.where(sel, uiv[pl.ds(g, LANES)], 0), axis=0)
            rp = jnp.sum(jnp.where(sel, piv[pl.ds(g, LANES)], 0), axis=0)
            rn = jnp.sum(jnp.where(sel, niv[pl.ds(g, LANES)], 0), axis=0)
            upar[slot, i, :] = jnp.full((LANES,), 0, jnp.int32) + (ru & 1)
            ppar[slot, i, :] = jnp.full((LANES,), 0, jnp.int32) + (rp & 1)
            npar[slot, i, :] = jnp.full((LANES,), 0, jnp.int32) + (rn & 1)
            pltpu.async_copy(utab_hbm.at[pl.ds(ru >> 1, 1), :],
                             ubuf.at[slot, pl.ds(i, 1), :], usem)
            pltpu.async_copy(etab_hbm.at[pl.ds(rp >> 1, 1), :],
                             pbuf.at[slot, pl.ds(i, 1), :], psem)
            pltpu.async_copy(etab_hbm.at[pl.ds(rn >> 1, 1), :],
                             nbuf.at[slot, pl.ds(i, 1), :], nsem)
            return 0
        lax.fori_loop(0, CH, enq, 0)

    def drain(slot):
        pltpu.make_async_copy(utab_hbm.at[pl.ds(0, CH), :],
                              ubuf.at[slot], usem).wait()
        pltpu.make_async_copy(etab_hbm.at[pl.ds(0, CH), :],
                              pbuf.at[slot], psem).wait()
        pltpu.make_async_copy(etab_hbm.at[pl.ds(0, CH), :],
                              nbuf.at[slot], nsem).wait()

    def chunk_sum(slot, acc):
        def row_body(i, a):
            mu = upar[slot, i, :] > 0
            mp = ppar[slot, i, :] > 0
            mn = npar[slot, i, :] > 0
            for j in range(DIM // LANES):
                sl0 = pl.ds(j * LANES, LANES)
                sl1 = pl.ds(DIM + j * LANES, LANES)
                u = jnp.where(mu, ubuf[slot, i, sl1], ubuf[slot, i, sl0])
                p = jnp.where(mp, pbuf[slot, i, sl1], pbuf[slot, i, sl0])
                n = jnp.where(mn, nbuf[slot, i, sl1], nbuf[slot, i, sl0])
                ps = u * p
                ns = u * n
                ea = jnp.exp(-jnp.abs(ps))
                eb = jnp.exp(-jnp.abs(ns))
                a = a + (jnp.maximum(-ps, 0.0) + jnp.maximum(ns, 0.0)
                         + _log1p_poly(ea) + _log1p_poly(eb))
            return a
        return lax.fori_loop(0, CH, row_body, acc)

    acc = jnp.zeros((LANES,), jnp.float32)
    fire(0, 0)
    for c in range(NCH):
        drain(c % 2)
        if c + 1 < NCH:
            fire(c + 1, (c + 1) % 2)
        acc = chunk_sum(c % 2, acc)

    part[...] = acc
    pltpu.sync_copy(part, out_hbm.at[wid])


@jax.jit
def _sc_partials(users, pos, neg, utab, etab):
    mesh = plsc.VectorSubcoreMesh(core_axis_name="c", subcore_axis_name="s")
    f = pl.kernel(
        _sc_body,
        out_type=jax.ShapeDtypeStruct((NW, LANES), jnp.float32),
        mesh=mesh,
        compiler_params=pltpu.CompilerParams(needs_layout_passes=False),
        scratch_types=[
            pltpu.VMEM((BPW,), jnp.int32),
            pltpu.VMEM((BPW,), jnp.int32),
            pltpu.VMEM((BPW,), jnp.int32),
            pltpu.VMEM((2, CH, ROWP), jnp.float32),
            pltpu.VMEM((2, CH, ROWP), jnp.float32),
            pltpu.VMEM((2, CH, ROWP), jnp.float32),
            pltpu.VMEM((2, CH, LANES), jnp.int32),
            pltpu.VMEM((2, CH, LANES), jnp.int32),
            pltpu.VMEM((2, CH, LANES), jnp.int32),
            pltpu.VMEM((LANES,), jnp.float32),
            pltpu.SemaphoreType.DMA,
            pltpu.SemaphoreType.DMA,
            pltpu.SemaphoreType.DMA,
        ],
    )
    return f(users, pos, neg, utab, etab)


def kernel(users, pos, neg, user_table, entity_table):
    utab2 = user_table.reshape(user_table.shape[0] // 2, ROWP)
    etab2 = entity_table.reshape(entity_table.shape[0] // 2, ROWP)
    parts = _sc_partials(users.astype(jnp.int32), pos.astype(jnp.int32),
                         neg.astype(jnp.int32), utab2, etab2)
    return jnp.sum(parts) / jnp.float32(B * DIM)


# speedup vs baseline: 4.2933x; 1.5803x over previous
import jax
import jax.numpy as jnp
from jax import lax
from jax.experimental import pallas as pl
from jax.experimental.pallas import tpu as pltpu
from jax.experimental.pallas import tpu_sc as plsc

B = 16384
DIM = 64
NC = 2
NS = 16
NW = NC * NS
BPW = B // NW
CH = 64
NCH = BPW // CH
LANES = 16

_LOG1P_COEF = (
    5.621959008883515e-07, 0.999957487075066, -0.49920656854784484,
    0.3269731000138668, -0.22283625832801954, 0.1307650325042385,
    -0.052624851367851076, 0.010119082927824848,
)


def _log1p_poly(t):
    acc = jnp.full_like(t, _LOG1P_COEF[-1])
    for c in reversed(_LOG1P_COEF[:-1]):
        acc = acc * t + jnp.float32(c)
    return acc


def _sc_body(users_hbm, pos_hbm, neg_hbm, utab_hbm, etab_hbm, out_hbm,
             uiv, piv, niv, ubuf, pbuf, nbuf, part,
             usem, psem, nsem):
    wid = lax.axis_index("s") * NC + lax.axis_index("c")
    base = wid * BPW

    pltpu.sync_copy(users_hbm.at[pl.ds(base, BPW)], uiv)
    pltpu.sync_copy(pos_hbm.at[pl.ds(base, BPW)], piv)
    pltpu.sync_copy(neg_hbm.at[pl.ds(base, BPW)], niv)

    lane = lax.iota(jnp.int32, LANES)

    def fire(c, slot):
        def enq(i, _):
            g = c * CH + (i & ~(LANES - 1))
            k = i & (LANES - 1)
            sel = lane == k
            ru = jnp.sum(jnp.where(sel, uiv[pl.ds(g, LANES)], 0), axis=0)
            rp = jnp.sum(jnp.where(sel, piv[pl.ds(g, LANES)], 0), axis=0)
            rn = jnp.sum(jnp.where(sel, niv[pl.ds(g, LANES)], 0), axis=0)
            pltpu.async_copy(utab_hbm.at[pl.ds(ru, 1), :],
                             ubuf.at[slot, pl.ds(i, 1), :], usem)
            pltpu.async_copy(etab_hbm.at[pl.ds(rp, 1), :],
                             pbuf.at[slot, pl.ds(i, 1), :], psem)
            pltpu.async_copy(etab_hbm.at[pl.ds(rn, 1), :],
                             nbuf.at[slot, pl.ds(i, 1), :], nsem)
            return 0
        lax.fori_loop(0, CH, enq, 0)

    def drain(slot):
        pltpu.make_async_copy(utab_hbm.at[pl.ds(0, CH), :],
                              ubuf.at[slot], usem).wait()
        pltpu.make_async_copy(etab_hbm.at[pl.ds(0, CH), :],
                              pbuf.at[slot], psem).wait()
        pltpu.make_async_copy(etab_hbm.at[pl.ds(0, CH), :],
                              nbuf.at[slot], nsem).wait()

    def chunk_sum(slot, acc):
        def row_body(i, a):
            for j in range(DIM // LANES):
                sl = pl.ds(j * LANES, LANES)
                u = ubuf[slot, i, sl]
                p = pbuf[slot, i, sl]
                n = nbuf[slot, i, sl]
                ps = u * p
                ns = u * n
                ea = jnp.exp(-jnp.abs(ps))
                eb = jnp.exp(-jnp.abs(ns))
                a = a + (jnp.maximum(-ps, 0.0) + jnp.maximum(ns, 0.0)
                         + _log1p_poly(ea) + _log1p_poly(eb))
            return a
        return lax.fori_loop(0, CH, row_body, acc)

    acc = jnp.zeros((LANES,), jnp.float32)
    fire(0, 0)
    for c in range(NCH):
        drain(c % 2)
        if c + 1 < NCH:
            fire(c + 1, (c + 1) % 2)
        acc = chunk_sum(c % 2, acc)

    part[...] = acc
    pltpu.sync_copy(part, out_hbm.at[wid])


@jax.jit
def _sc_partials(users, pos, neg, utab, etab):
    mesh = plsc.VectorSubcoreMesh(core_axis_name="c", subcore_axis_name="s")
    f = pl.kernel(
        _sc_body,
        out_type=jax.ShapeDtypeStruct((NW, LANES), jnp.float32),
        mesh=mesh,
        compiler_params=pltpu.CompilerParams(needs_layout_passes=False),
        scratch_types=[
            pltpu.VMEM((BPW,), jnp.int32),
            pltpu.VMEM((BPW,), jnp.int32),
            pltpu.VMEM((BPW,), jnp.int32),
            pltpu.VMEM((2, CH, DIM), jnp.float32),
            pltpu.VMEM((2, CH, DIM), jnp.float32),
            pltpu.VMEM((2, CH, DIM), jnp.float32),
            pltpu.VMEM((LANES,), jnp.float32),
            pltpu.SemaphoreType.DMA,
            pltpu.SemaphoreType.DMA,
            pltpu.SemaphoreType.DMA,
        ],
    )
    return f(users, pos, neg, utab, etab)


def kernel(users, pos, neg, user_table, entity_table):
    parts = _sc_partials(users.astype(jnp.int32), pos.astype(jnp.int32),
                         neg.astype(jnp.int32), user_table, entity_table)
    return jnp.sum(parts) / jnp.float32(B * DIM)
